# baseline (speedup 1.0000x reference)
import functools

import jax
import jax.numpy as jnp
from jax import lax
from jax.experimental import pallas as pl
from jax.experimental.pallas import tpu as pltpu
from jax.experimental.pallas import tpu_sc as plsc

V = 100000
H = 768
S = 512
B = 32
N_TOK = B * S
N_ELEM = float(N_TOK * H)


def _sc_gather_kernel():
    info = plsc.get_sparse_core_info()
    nc, ns = info.num_cores, info.num_subcores
    nw = nc * ns
    tpw = N_TOK // nw
    ch = 32
    nch = tpw // ch
    nbuf = 5
    mesh = plsc.VectorSubcoreMesh(core_axis_name="c", subcore_axis_name="s")

    @functools.partial(
        pl.kernel,
        out_type=jax.ShapeDtypeStruct((N_TOK, H), jnp.float32),
        mesh=mesh,
        scratch_types=[
            pltpu.VMEM((tpw,), jnp.int32),
            pltpu.VMEM((nbuf, ch, H), jnp.float32),
        ]
        + [pltpu.SemaphoreType.DMA] * (2 * nbuf),
    )
    def sc_kernel(ids_hbm, word_hbm, raw_hbm, idsbuf, rbuf, *sems):
        gsems, wsems = sems[:nbuf], sems[nbuf:]
        wid = lax.axis_index("s") * nc + lax.axis_index("c")
        base = wid * tpw
        pltpu.sync_copy(ids_hbm.at[pl.ds(base, tpw)], idsbuf)

        def start_gather(j):
            k = j % nbuf
            pltpu.async_copy(
                word_hbm.at[idsbuf.at[pl.ds(j * ch, ch)]], rbuf.at[k], gsems[k]
            )

        def wait_gather(j):
            k = j % nbuf
            pltpu.make_async_copy(
                word_hbm.at[idsbuf.at[pl.ds(j * ch, ch)]], rbuf.at[k], gsems[k]
            ).wait()

        def start_write(j):
            k = j % nbuf
            pltpu.async_copy(
                rbuf.at[k], raw_hbm.at[pl.ds(base + j * ch, ch), :], wsems[k]
            )

        def wait_write(j):
            k = j % nbuf
            pltpu.make_async_copy(
                rbuf.at[k], raw_hbm.at[pl.ds(base + j * ch, ch), :], wsems[k]
            ).wait()

        for j in range(min(nbuf, nch)):
            start_gather(j)
        for j in range(nch):
            wait_gather(j)
            start_write(j)
            if j + nbuf < nch:
                wait_write(j)
                start_gather(j + nbuf)
        for j in range(max(0, nch - nbuf), nch):
            wait_write(j)

    return sc_kernel


def _tc_norm(raw, ttcol, pos0, dty):
    blk = 512
    nb = N_TOK // blk

    def body(raw_ref, tt_ref, pos_ref, ty_ref, o_ref, acc, xbuf, psum):
        p = pl.program_id(0)
        i = pl.program_id(1)
        ones = jnp.ones((8, blk), jnp.float32)

        @pl.when((p == 0) & (i == 0))
        def _init():
            psum[...] = jnp.zeros_like(psum)

        @pl.when(p == 0)
        def _accum():
            x = (
                raw_ref[...]
                + pos_ref[...]
                + jnp.dot(tt_ref[...], ty_ref[...],
                          precision=lax.Precision.HIGHEST,
                          preferred_element_type=jnp.float32)
            )
            xbuf[pl.ds(i * blk, blk), :] = x
            psum[0] += jnp.dot(ones, x,
                               precision=lax.Precision.HIGHEST,
                               preferred_element_type=jnp.float32)
            psum[1] += jnp.dot(ones, x * x,
                               precision=lax.Precision.HIGHEST,
                               preferred_element_type=jnp.float32)

        @pl.when((p == 1) & (i == 0))
        def _finalize():
            mean = jnp.sum(psum[0, 0, :]) / N_ELEM
            var = jnp.sum(psum[1, 0, :]) / N_ELEM - mean * mean
            acc[0] = mean
            acc[1] = lax.rsqrt(var + 1e-5)

        @pl.when(p == 1)
        def _norm():
            x = xbuf[pl.ds(i * blk, blk), :]
            o_ref[...] = (x - acc[0]) * acc[1]

    return pl.pallas_call(
        body,
        grid=(2, nb),
        in_specs=[
            pl.BlockSpec((blk, H), lambda p, i: (i * (1 - p), 0)),
            pl.BlockSpec((blk, 1), lambda p, i: (i * (1 - p), 0)),
            pl.BlockSpec((S, H), lambda p, i: (0, 0)),
            pl.BlockSpec((1, H), lambda p, i: (0, 0)),
        ],
        out_specs=pl.BlockSpec((blk, H), lambda p, i: (i * p, 0)),
        out_shape=jax.ShapeDtypeStruct((N_TOK, H), jnp.float32),
        scratch_shapes=[
            pltpu.SMEM((2,), jnp.float32),
            pltpu.VMEM((N_TOK, H), jnp.float32),
            pltpu.VMEM((2, 8, H), jnp.float32),
        ],
        compiler_params=pltpu.CompilerParams(
            vmem_limit_bytes=62 * 1024 * 1024,
        ),
    )(raw, ttcol, pos0, dty)


def kernel(input_ids, token_type_ids, word_emb, type_emb, pos_emb):
    ids = input_ids.reshape(-1).astype(jnp.int32)
    raw = _sc_gather_kernel()(ids, word_emb)
    ttcol = token_type_ids.reshape(-1, 1).astype(jnp.float32)
    pos0 = pos_emb + type_emb[0]
    dty = (type_emb[1] - type_emb[0]).reshape(1, H)
    out = _tc_norm(raw, ttcol, pos0, dty)
    return out.reshape(B, S, H)


# speedup vs baseline: 1.2143x; 1.2143x over previous
import functools

import jax
import jax.numpy as jnp
from jax import lax
from jax.experimental import pallas as pl
from jax.experimental.pallas import tpu as pltpu
from jax.experimental.pallas import tpu_sc as plsc

V = 100000
H = 768
S = 512
B = 32
N_TOK = B * S
N_ELEM = float(N_TOK * H)


def _sc_gather_kernel():
    info = plsc.get_sparse_core_info()
    nc, ns = info.num_cores, info.num_subcores
    nw = nc * ns
    tpw = N_TOK // nw
    ch = 32
    nch = tpw // ch
    nbuf = 5
    mesh = plsc.VectorSubcoreMesh(core_axis_name="c", subcore_axis_name="s")

    @functools.partial(
        pl.kernel,
        out_type=jax.ShapeDtypeStruct((N_TOK, H), jnp.float32),
        mesh=mesh,
        scratch_types=[
            pltpu.VMEM((tpw,), jnp.int32),
            pltpu.VMEM((nbuf, ch, H), jnp.float32),
        ]
        + [pltpu.SemaphoreType.DMA] * (2 * nbuf),
    )
    def sc_kernel(ids_hbm, word_hbm, raw_hbm, idsbuf, rbuf, *sems):
        gsems, wsems = sems[:nbuf], sems[nbuf:]
        wid = lax.axis_index("s") * nc + lax.axis_index("c")
        base = wid * tpw
        pltpu.sync_copy(ids_hbm.at[pl.ds(base, tpw)], idsbuf)

        def start_gather(j):
            k = j % nbuf
            pltpu.async_copy(
                word_hbm.at[idsbuf.at[pl.ds(j * ch, ch)]], rbuf.at[k], gsems[k]
            )

        def wait_gather(j):
            k = j % nbuf
            pltpu.make_async_copy(
                word_hbm.at[idsbuf.at[pl.ds(j * ch, ch)]], rbuf.at[k], gsems[k]
            ).wait()

        def start_write(j):
            k = j % nbuf
            pltpu.async_copy(
                rbuf.at[k], raw_hbm.at[pl.ds(base + j * ch, ch), :], wsems[k]
            )

        def wait_write(j):
            k = j % nbuf
            pltpu.make_async_copy(
                rbuf.at[k], raw_hbm.at[pl.ds(base + j * ch, ch), :], wsems[k]
            ).wait()

        for j in range(min(nbuf, nch)):
            start_gather(j)
        for j in range(nch):
            wait_gather(j)
            start_write(j)
            if j + nbuf < nch:
                wait_write(j)
                start_gather(j + nbuf)
        for j in range(max(0, nch - nbuf), nch):
            wait_write(j)

    return sc_kernel


def _tc_norm(raw, ttcol, pos0, dty):
    blk = 512
    nb = N_TOK // blk

    def body(raw_ref, tt_ref, pos_ref, ty_ref, o_ref, acc, xbuf):
        p = pl.program_id(0)
        i = pl.program_id(1)

        @pl.when((p == 0) & (i == 0))
        def _init():
            acc[0] = 0.0
            acc[1] = 0.0

        @pl.when(p == 0)
        def _accum():
            x = (
                raw_ref[...]
                + pos_ref[...]
                + jnp.dot(tt_ref[...], ty_ref[...],
                          precision=lax.Precision.HIGHEST,
                          preferred_element_type=jnp.float32)
            )
            xbuf[pl.ds(i * blk, blk), :] = x
            acc[0] += jnp.sum(x)
            acc[1] += jnp.sum(x * x)

        @pl.when(p == 1)
        def _norm():
            mean = acc[0] / N_ELEM
            var = acc[1] / N_ELEM - mean * mean
            x = xbuf[pl.ds(i * blk, blk), :]
            o_ref[...] = (x - mean) * lax.rsqrt(var + 1e-5)

    return pl.pallas_call(
        body,
        grid=(2, nb),
        in_specs=[
            pl.BlockSpec((blk, H), lambda p, i: (i * (1 - p), 0)),
            pl.BlockSpec((blk, 1), lambda p, i: (i * (1 - p), 0)),
            pl.BlockSpec((S, H), lambda p, i: (0, 0)),
            pl.BlockSpec((1, H), lambda p, i: (0, 0)),
        ],
        out_specs=pl.BlockSpec((blk, H), lambda p, i: (i * p, 0)),
        out_shape=jax.ShapeDtypeStruct((N_TOK, H), jnp.float32),
        scratch_shapes=[
            pltpu.SMEM((2,), jnp.float32),
            pltpu.VMEM((N_TOK, H), jnp.float32),
        ],
    )(raw, ttcol, pos0, dty)


def kernel(input_ids, token_type_ids, word_emb, type_emb, pos_emb):
    ids = input_ids.reshape(-1).astype(jnp.int32)
    raw = _sc_gather_kernel()(ids, word_emb)
    ttcol = token_type_ids.reshape(-1, 1).astype(jnp.float32)
    pos0 = pos_emb + type_emb[0]
    dty = (type_emb[1] - type_emb[0]).reshape(1, H)
    out = _tc_norm(raw, ttcol, pos0, dty)
    return out.reshape(B, S, H)


# speedup vs baseline: 1.4559x; 1.1990x over previous
import functools

import jax
import jax.numpy as jnp
from jax import lax
from jax.experimental import pallas as pl
from jax.experimental.pallas import tpu as pltpu
from jax.experimental.pallas import tpu_sc as plsc

V = 100000
H = 768
S = 512
B = 32
N_TOK = B * S
N_ELEM = float(N_TOK * H)


def _sc_gather_kernel():
    info = plsc.get_sparse_core_info()
    nc, ns = info.num_cores, info.num_subcores
    nw = nc * ns
    tpw = N_TOK // nw
    ch = 32
    nch = tpw // ch
    nbuf = 5
    mesh = plsc.VectorSubcoreMesh(core_axis_name="c", subcore_axis_name="s")

    @functools.partial(
        pl.kernel,
        out_type=jax.ShapeDtypeStruct((N_TOK, H), jnp.float32),
        mesh=mesh,
        scratch_types=[
            pltpu.VMEM((tpw,), jnp.int32),
            pltpu.VMEM((nbuf, ch, H), jnp.float32),
        ]
        + [pltpu.SemaphoreType.DMA] * (2 * nbuf),
    )
    def sc_kernel(ids_hbm, word_hbm, raw_hbm, idsbuf, rbuf, *sems):
        gsems, wsems = sems[:nbuf], sems[nbuf:]
        wid = lax.axis_index("s") * nc + lax.axis_index("c")
        base = wid * tpw
        pltpu.sync_copy(ids_hbm.at[pl.ds(base, tpw)], idsbuf)

        def start_gather(j):
            k = j % nbuf
            pltpu.async_copy(
                word_hbm.at[idsbuf.at[pl.ds(j * ch, ch)]], rbuf.at[k], gsems[k]
            )

        def wait_gather(j):
            k = j % nbuf
            pltpu.make_async_copy(
                word_hbm.at[idsbuf.at[pl.ds(j * ch, ch)]], rbuf.at[k], gsems[k]
            ).wait()

        def start_write(j):
            k = j % nbuf
            pltpu.async_copy(
                rbuf.at[k], raw_hbm.at[pl.ds(base + j * ch, ch), :], wsems[k]
            )

        def wait_write(j):
            k = j % nbuf
            pltpu.make_async_copy(
                rbuf.at[k], raw_hbm.at[pl.ds(base + j * ch, ch), :], wsems[k]
            ).wait()

        for j in range(min(nbuf, nch)):
            start_gather(j)
        for j in range(nch):
            wait_gather(j)
            start_write(j)
            if j + nbuf < nch:
                wait_write(j)
                start_gather(j + nbuf)
        for j in range(max(0, nch - nbuf), nch):
            wait_write(j)

    return sc_kernel


def _tc_norm(raw, ttcol, pos0, dty):
    blk = 512
    nb = N_TOK // blk

    def body(raw_ref, tt_ref, pos_ref, ty_ref, o_ref, acc, xbuf):
        p = pl.program_id(0)
        i = pl.program_id(1)

        @pl.when((p == 0) & (i == 0))
        def _init():
            acc[0] = 0.0
            acc[1] = 0.0

        @pl.when(p == 0)
        def _accum():
            x = raw_ref[...] + pos_ref[...] + tt_ref[...] * ty_ref[...]
            xbuf[pl.ds(i * blk, blk), :] = x
            acc[0] += jnp.sum(x)
            acc[1] += jnp.sum(x * x)

        @pl.when(p == 1)
        def _norm():
            mean = acc[0] / N_ELEM
            var = acc[1] / N_ELEM - mean * mean
            x = xbuf[pl.ds(i * blk, blk), :]
            o_ref[...] = (x - mean) * lax.rsqrt(var + 1e-5)

    return pl.pallas_call(
        body,
        grid=(2, nb),
        in_specs=[
            pl.BlockSpec((blk, H), lambda p, i: (i * (1 - p), 0)),
            pl.BlockSpec((blk, 1), lambda p, i: (i * (1 - p), 0)),
            pl.BlockSpec((S, H), lambda p, i: (0, 0)),
            pl.BlockSpec((1, H), lambda p, i: (0, 0)),
        ],
        out_specs=pl.BlockSpec((blk, H), lambda p, i: (i * p, 0)),
        out_shape=jax.ShapeDtypeStruct((N_TOK, H), jnp.float32),
        scratch_shapes=[
            pltpu.SMEM((2,), jnp.float32),
            pltpu.VMEM((N_TOK, H), jnp.float32),
        ],
    )(raw, ttcol, pos0, dty)


def kernel(input_ids, token_type_ids, word_emb, type_emb, pos_emb):
    ids = input_ids.reshape(-1).astype(jnp.int32)
    raw = _sc_gather_kernel()(ids, word_emb)
    ttcol = token_type_ids.reshape(-1, 1).astype(jnp.float32)
    pos0 = pos_emb + type_emb[0]
    dty = (type_emb[1] - type_emb[0]).reshape(1, H)
    out = _tc_norm(raw, ttcol, pos0, dty)
    return out.reshape(B, S, H)
